# Initial kernel scaffold; baseline (speedup 1.0000x reference)
import jax
import jax.numpy as jnp
from jax.experimental import pallas as pl


def kernel(x, edge_index, Wl0, bl0, Wr0, Wl1, bl1, Wr1, Wl2, bl2, Wr2):
    raise NotImplementedError("write your pallas kernel here")


# speedup vs baseline: 1.0000x; 1.0000x over previous
import jax, jax.numpy as jnp
from jax.experimental import pallas as pl

N = 10000


def _layer(x, src, dst, Wl, bl, Wr):
    msg = jnp.take(x, src, axis=0)
    summed = jax.ops.segment_sum(msg, dst, num_segments=N)
    cnt = jax.ops.segment_sum(jnp.ones((src.shape[0],), dtype=x.dtype), dst, num_segments=N)
    mean = summed / jnp.clip(cnt, 1.0, None)[:, None]
    return mean @ Wl.T + bl + x @ Wr.T


def kernel(x, edge_index, Wl0, bl0, Wr0, Wl1, bl1, Wr1, Wl2, bl2, Wr2):
    src = edge_index[0]
    dst = edge_index[1]
    h = jax.nn.relu(_layer(x, src, dst, Wl0, bl0, Wr0))
    h = jax.nn.relu(_layer(h, src, dst, Wl1, bl1, Wr1))
    h = _layer(h, src, dst, Wl2, bl2, Wr2)
    return (h, jax.nn.sigmoid(h))


# speedup vs baseline: 5.5204x; 5.5203x over previous
import jax
import jax.numpy as jnp
from jax import lax
from jax.experimental import pallas as pl
from jax.experimental.pallas import tpu as pltpu
from jax.experimental.pallas import tpu_sc as plsc

N = 10000
NP = 10240
E = 160000
NC = 2
NS = 16
NB = 40
EB = 125
RPT = NP // NS
WB = 128
RJ = RPT // WB
F = 128
BM = 1024
MB = NP // BM


def _sc_mesh():
    return plsc.VectorSubcoreMesh(core_axis_name="c", subcore_axis_name="s")


def _make_agg(C: int):
    out_type = jax.ShapeDtypeStruct((NC, C, NP, F), jnp.float32)

    scratch = [
        pltpu.VMEM_SHARED((NP, F), jnp.float32),
        pltpu.VMEM((NB, EB), jnp.int32),
        pltpu.VMEM((NB, EB), jnp.int32),
        pltpu.VMEM((EB, F), jnp.float32),
        pltpu.VMEM((EB, F), jnp.float32),
        pltpu.SemaphoreType.DMA,
        pltpu.SemaphoreType.DMA,
    ]

    def body(h_flat, src_off, dst_r, zrows, s_out, accum, srcv, dstv,
             rows0, rows1, sem0, sem1):
        ci = lax.axis_index("c")
        si = lax.axis_index("s")
        tbase = si * RPT

        pltpu.sync_copy(dst_r.at[ci, si], dstv)

        def zero_accum():
            pltpu.sync_copy(zrows, accum.at[pl.ds(tbase, RPT)])

        def writeout(dst_hbm):
            pltpu.sync_copy(accum.at[pl.ds(tbase, RPT)],
                            dst_hbm.at[pl.ds(tbase, RPT)])

        def gather(b, rbuf, sem):
            pltpu.async_copy(h_flat.at[srcv.at[b]], rbuf, sem)

        def gwait(b, rbuf, sem):
            pltpu.make_async_copy(h_flat.at[srcv.at[b]], rbuf, sem).wait()

        def chunk_pass(c, _):
            pltpu.sync_copy(src_off.at[c, ci, si], srcv)
            zero_accum()
            plsc.subcore_barrier()

            gather(0, rows0, sem0)

            def pair(g, __):
                b0 = 2 * g
                b1 = 2 * g + 1
                gwait(b0, rows0, sem0)
                gather(b1, rows1, sem1)
                pltpu.sync_copy(rows0, accum.at[dstv.at[b0]], add=True)
                gwait(b1, rows1, sem1)

                @pl.when(g < NB // 2 - 1)
                def _():
                    gather(b0 + 2, rows0, sem0)

                pltpu.sync_copy(rows1, accum.at[dstv.at[b1]], add=True)
                return 0

            lax.fori_loop(0, NB // 2, pair, 0)
            plsc.subcore_barrier()
            writeout(s_out.at[ci, c])
            plsc.subcore_barrier()
            return 0

        lax.fori_loop(0, C, chunk_pass, 0)

    return pl.kernel(
        body,
        out_type=out_type,
        mesh=_sc_mesh(),
        scratch_types=scratch,
    )


def _make_mm(C_a: int, C_h: int, C_out: int, act, has_bias: bool):
    K = C_a + C_h
    DOUT = C_out * F

    def body(*refs):
        it = iter(refs)
        if C_a:
            s_ref = next(it)
            cnt_ref = next(it)
        h_ref = next(it)
        wl_ref = next(it) if C_a else None
        wr_ref = next(it)
        b_ref = next(it) if has_bias else None
        out_ref = next(it)

        k = pl.program_id(1)

        @pl.when(k == 0)
        def _():
            out_ref[...] = jnp.zeros(out_ref.shape, out_ref.dtype)

        def accum(t):
            for c in range(C_out):
                out_ref[c] += t[:, c * F:(c + 1) * F]

        if C_a:
            @pl.when(k < C_a)
            def _():
                cnt = cnt_ref[0, :, 0:1] + cnt_ref[1, :, 0:1]
                inv = 1.0 / jnp.maximum(cnt, 1.0)
                a = (s_ref[0, 0] + s_ref[1, 0]) * inv
                accum(jnp.dot(a, wl_ref[0],
                              preferred_element_type=jnp.float32))

        @pl.when(k >= C_a)
        def _():
            accum(jnp.dot(h_ref[0], wr_ref[0],
                          preferred_element_type=jnp.float32))

        @pl.when(k == K - 1)
        def _():
            for c in range(C_out):
                v = out_ref[c]
                if has_bias:
                    v = v + b_ref[0, c * F:(c + 1) * F][None, :]
                if act is not None:
                    v = act(v)
                out_ref[c] = v

    in_specs = []
    if C_a:
        in_specs.append(pl.BlockSpec(
            (NC, 1, BM, F), lambda m, k: (0, jnp.minimum(k, C_a - 1), m, 0)))
        in_specs.append(pl.BlockSpec((NC, BM, F), lambda m, k: (0, m, 0)))
    in_specs.append(pl.BlockSpec(
        (1, BM, F), lambda m, k: (jnp.maximum(k - C_a, 0), m, 0)))
    if C_a:
        in_specs.append(pl.BlockSpec(
            (1, F, DOUT), lambda m, k: (jnp.minimum(k, C_a - 1), 0, 0)))
    in_specs.append(pl.BlockSpec(
        (1, F, DOUT), lambda m, k: (jnp.maximum(k - C_a, 0), 0, 0)))
    if has_bias:
        in_specs.append(pl.BlockSpec((1, DOUT), lambda m, k: (0, 0)))

    return pl.pallas_call(
        body,
        grid=(MB, K),
        in_specs=in_specs,
        out_specs=pl.BlockSpec((C_out, BM, F), lambda m, k: (0, m, 0)),
        out_shape=jax.ShapeDtypeStruct((C_out, NP, F), jnp.float32),
        compiler_params=pltpu.CompilerParams(
            dimension_semantics=("parallel", "arbitrary")),
    )


def _make_final(C_h: int):
    K = C_h
    DOUT = 2 * F

    def body(sp_ref, cnt_ref, h_ref, wr_ref, b_ref, o1_ref, o2_ref):
        k = pl.program_id(1)

        @pl.when(k == 0)
        def _():
            o1_ref[...] = jnp.zeros(o1_ref.shape, o1_ref.dtype)

        o1_ref[...] += jnp.dot(h_ref[0], wr_ref[0],
                               preferred_element_type=jnp.float32)

        @pl.when(k == K - 1)
        def _():
            cnt = cnt_ref[0, :, 0:1] + cnt_ref[1, :, 0:1]
            inv = 1.0 / jnp.maximum(cnt, 1.0)
            mean = jnp.concatenate(
                [(sp_ref[0, c] + sp_ref[1, c]) * inv for c in range(2)],
                axis=1)
            v = o1_ref[...] + mean + b_ref[0][None, :]
            o1_ref[...] = v
            o2_ref[...] = jax.nn.sigmoid(v)

    return pl.pallas_call(
        body,
        grid=(MB, K),
        in_specs=[
            pl.BlockSpec((NC, 2, BM, F), lambda m, k: (0, 0, m, 0)),
            pl.BlockSpec((NC, BM, F), lambda m, k: (0, m, 0)),
            pl.BlockSpec((1, BM, F), lambda m, k: (k, m, 0)),
            pl.BlockSpec((1, F, DOUT), lambda m, k: (k, 0, 0)),
            pl.BlockSpec((1, DOUT), lambda m, k: (0, 0)),
        ],
        out_specs=[
            pl.BlockSpec((BM, DOUT), lambda m, k: (m, 0)),
            pl.BlockSpec((BM, DOUT), lambda m, k: (m, 0)),
        ],
        out_shape=[
            jax.ShapeDtypeStruct((NP, DOUT), jnp.float32),
            jax.ShapeDtypeStruct((NP, DOUT), jnp.float32),
        ],
        compiler_params=pltpu.CompilerParams(
            dimension_semantics=("parallel", "arbitrary")),
    )


def _chunk_w(w):
    dout, din = w.shape
    return w.T.reshape(din // F, F, dout)


def _src_offsets(src_r, C):
    return (src_r[None] +
            (jnp.arange(C, dtype=jnp.int32) * NP).reshape(C, 1, 1, 1, 1))


def kernel(x, edge_index, Wl0, bl0, Wr0, Wl1, bl1, Wr1, Wl2, bl2, Wr2):
    src = edge_index[0].reshape(NC, NS, NB, EB)
    dst = edge_index[1].reshape(NC, NS, NB, EB)

    xp = jnp.pad(x, ((0, NP - N), (0, 0)))
    x_cm = jnp.stack([xp[:, :F], xp[:, F:]])

    agg3 = _make_agg(3)
    agg4 = _make_agg(4)
    agg2 = _make_agg(2)

    so2 = _src_offsets(src, 2)
    so3 = _src_offsets(src, 3)
    so4 = _src_offsets(src, 4)
    zrows = jnp.zeros((RPT, F), jnp.float32)

    h_flat0 = jnp.concatenate(
        [x_cm.reshape(2 * NP, F), jnp.ones((NP, F), jnp.float32)])
    s_all = agg3(h_flat0, so3, dst, zrows)
    s_x = s_all[:, :2]
    cnt = s_all[:, 2]
    h1 = _make_mm(2, 2, 4, jax.nn.relu, True)(
        s_x, cnt, x_cm, _chunk_w(Wl0), _chunk_w(Wr0), bl0.reshape(1, -1))

    s_1 = agg4(h1.reshape(4 * NP, F), so4, dst, zrows)
    h2 = _make_mm(4, 4, 4, jax.nn.relu, True)(
        s_1, cnt, h1, _chunk_w(Wl1), _chunk_w(Wr1), bl1.reshape(1, -1))

    p = _make_mm(0, 4, 2, None, False)(h2, _chunk_w(Wl2))
    s_p = agg2(p.reshape(2 * NP, F), so2, dst, zrows)
    h, sig = _make_final(4)(s_p, cnt, h2, _chunk_w(Wr2), bl2.reshape(1, -1))
    return (h[:N], sig[:N])


# speedup vs baseline: 6.2444x; 1.1312x over previous
import jax
import jax.numpy as jnp
from jax import lax
from jax.experimental import pallas as pl
from jax.experimental.pallas import tpu as pltpu
from jax.experimental.pallas import tpu_sc as plsc

N = 10000
NP = 10240
E = 160000
NC = 2
NS = 16
NB = 40
EB = 125
RPT = NP // NS
F = 128
BM = 1024
MB = NP // BM


def _sc_mesh():
    return plsc.VectorSubcoreMesh(core_axis_name="c", subcore_axis_name="s")


def _make_agg(C: int):
    out_type = jax.ShapeDtypeStruct((NC, C, NP, F), jnp.float32)

    scratch = [
        pltpu.VMEM_SHARED((NP, F), jnp.float32),
        pltpu.VMEM((NB, EB), jnp.int32),
        pltpu.VMEM((NB, EB), jnp.int32),
        pltpu.VMEM((EB, F), jnp.float32),
        pltpu.VMEM((EB, F), jnp.float32),
        pltpu.SemaphoreType.DMA,
        pltpu.SemaphoreType.DMA,
    ]

    def body(h_flat, src_off, dst_r, zrows, s_out, accum, srcv, dstv,
             rows0, rows1, sem0, sem1):
        ci = lax.axis_index("c")
        si = lax.axis_index("s")
        tbase = si * RPT

        pltpu.sync_copy(dst_r.at[ci, si], dstv)

        def gather(b, rbuf, sem):
            pltpu.async_copy(h_flat.at[srcv.at[b]], rbuf, sem)

        def gwait(b, rbuf, sem):
            pltpu.make_async_copy(h_flat.at[srcv.at[b]], rbuf, sem).wait()

        def chunk_pass(c, _):
            pltpu.sync_copy(src_off.at[c, ci, si], srcv)
            pltpu.sync_copy(zrows, accum.at[pl.ds(tbase, RPT)])
            plsc.subcore_barrier()

            gather(0, rows0, sem0)

            def pair(g, __):
                b0 = 2 * g
                b1 = 2 * g + 1
                gwait(b0, rows0, sem0)
                gather(b1, rows1, sem1)
                pltpu.sync_copy(rows0, accum.at[dstv.at[b0]], add=True)
                gwait(b1, rows1, sem1)

                @pl.when(g < NB // 2 - 1)
                def _():
                    gather(b0 + 2, rows0, sem0)

                pltpu.sync_copy(rows1, accum.at[dstv.at[b1]], add=True)
                return 0

            lax.fori_loop(0, NB // 2, pair, 0)
            plsc.subcore_barrier()
            pltpu.sync_copy(accum.at[pl.ds(tbase, RPT)],
                            s_out.at[ci, c, pl.ds(tbase, RPT)])
            plsc.subcore_barrier()
            return 0

        lax.fori_loop(0, C, chunk_pass, 0)

    return pl.kernel(
        body,
        out_type=out_type,
        mesh=_sc_mesh(),
        scratch_types=scratch,
    )


_TC_PARAMS = dict(
    compiler_params=pltpu.CompilerParams(
        dimension_semantics=("parallel", "arbitrary")),
)


def _make_wr(C_h: int, C_out: int):
    K = C_h
    DOUT = C_out * F

    def body(h_ref, wr_ref, b_ref, out_ref):
        k = pl.program_id(1)

        @pl.when(k == 0)
        def _():
            out_ref[...] = jnp.zeros(out_ref.shape, out_ref.dtype)

        t = jnp.dot(h_ref[0], wr_ref[0], preferred_element_type=jnp.float32)
        for c in range(C_out):
            out_ref[c] += t[:, c * F:(c + 1) * F]

        @pl.when(k == K - 1)
        def _():
            for c in range(C_out):
                out_ref[c] += b_ref[0, c * F:(c + 1) * F][None, :]

    return pl.pallas_call(
        body,
        grid=(MB, K),
        in_specs=[
            pl.BlockSpec((1, BM, F), lambda m, k: (k, m, 0)),
            pl.BlockSpec((1, F, DOUT), lambda m, k: (k, 0, 0)),
            pl.BlockSpec((1, DOUT), lambda m, k: (0, 0)),
        ],
        out_specs=pl.BlockSpec((C_out, BM, F), lambda m, k: (0, m, 0)),
        out_shape=jax.ShapeDtypeStruct((C_out, NP, F), jnp.float32),
        **_TC_PARAMS,
    )


def _make_combine(C_a: int, C_out: int, cnt_cs: int, cnt_idx: int,
                  sec_dout: int, act, emit_h: bool):
    K = C_a
    DOUT = C_out * F
    SC_CH = sec_dout // F if sec_dout else 0

    def body(*refs):
        it = iter(refs)
        s_ref = next(it)
        cnt_ref = next(it)
        wrp_ref = next(it)
        wl_ref = next(it)
        w2_ref = next(it) if sec_dout else None
        b2_ref = next(it) if sec_dout else None
        h_ref = next(it) if emit_h else None
        sec_ref = next(it) if sec_dout else None
        acc_ref = next(it)

        k = pl.program_id(1)

        @pl.when(k == 0)
        def _():
            acc_ref[...] = jnp.zeros(acc_ref.shape, acc_ref.dtype)

        cnt = cnt_ref[0, 0, :, 0:1] + cnt_ref[1, 0, :, 0:1]
        inv = 1.0 / jnp.maximum(cnt, 1.0)
        a = (s_ref[0, 0] + s_ref[1, 0]) * inv
        acc_ref[...] += jnp.dot(a, wl_ref[0],
                                preferred_element_type=jnp.float32)

        @pl.when(k == K - 1)
        def _():
            parts = []
            for c in range(C_out):
                v = acc_ref[:, c * F:(c + 1) * F] + wrp_ref[c]
                if act is not None:
                    v = act(v)
                if emit_h:
                    h_ref[c] = v
                parts.append(v)
            if sec_dout:
                vfull = jnp.concatenate(parts, axis=1)
                t2 = jnp.dot(vfull, w2_ref[...],
                             preferred_element_type=jnp.float32)
                for c2 in range(SC_CH):
                    sec_ref[c2] = (t2[:, c2 * F:(c2 + 1) * F]
                                   + b2_ref[0, c2 * F:(c2 + 1) * F][None, :])

    in_specs = [
        pl.BlockSpec((NC, 1, BM, F), lambda m, k: (0, k, m, 0)),
        pl.BlockSpec((NC, 1, BM, F), lambda m, k: (0, cnt_idx, m, 0)),
        pl.BlockSpec((C_out, BM, F), lambda m, k: (0, m, 0)),
        pl.BlockSpec((1, F, DOUT), lambda m, k: (k, 0, 0)),
    ]
    if sec_dout:
        in_specs.append(pl.BlockSpec((DOUT, sec_dout), lambda m, k: (0, 0)))
        in_specs.append(pl.BlockSpec((1, sec_dout), lambda m, k: (0, 0)))

    out_specs = []
    out_shape = []
    if emit_h:
        out_specs.append(pl.BlockSpec((C_out, BM, F), lambda m, k: (0, m, 0)))
        out_shape.append(jax.ShapeDtypeStruct((C_out, NP, F), jnp.float32))
    if sec_dout:
        out_specs.append(pl.BlockSpec((SC_CH, BM, F), lambda m, k: (0, m, 0)))
        out_shape.append(jax.ShapeDtypeStruct((SC_CH, NP, F), jnp.float32))

    return pl.pallas_call(
        body,
        grid=(MB, K),
        in_specs=in_specs,
        out_specs=out_specs,
        out_shape=out_shape,
        scratch_shapes=[pltpu.VMEM((BM, DOUT), jnp.float32)],
        **_TC_PARAMS,
    )


def _make_final(cnt_cs: int, cnt_idx: int):
    DOUT = 2 * F

    def body(sp_ref, cnt_ref, w2_ref, o1_ref, o2_ref):
        cnt = cnt_ref[0, 0, :, 0:1] + cnt_ref[1, 0, :, 0:1]
        inv = 1.0 / jnp.maximum(cnt, 1.0)
        mean = jnp.concatenate(
            [(sp_ref[0, c] + sp_ref[1, c]) * inv for c in range(2)], axis=1)
        wrp2 = jnp.concatenate([w2_ref[c] for c in range(2)], axis=1)
        v = mean + wrp2
        o1_ref[...] = v
        o2_ref[...] = jax.nn.sigmoid(v)

    return pl.pallas_call(
        body,
        grid=(MB,),
        in_specs=[
            pl.BlockSpec((NC, 2, BM, F), lambda m: (0, 0, m, 0)),
            pl.BlockSpec((NC, 1, BM, F), lambda m: (0, cnt_idx, m, 0)),
            pl.BlockSpec((2, BM, F), lambda m: (1, m, 0)),
        ],
        out_specs=[
            pl.BlockSpec((BM, DOUT), lambda m: (m, 0)),
            pl.BlockSpec((BM, DOUT), lambda m: (m, 0)),
        ],
        out_shape=[
            jax.ShapeDtypeStruct((NP, DOUT), jnp.float32),
            jax.ShapeDtypeStruct((NP, DOUT), jnp.float32),
        ],
        compiler_params=pltpu.CompilerParams(
            dimension_semantics=("parallel",)),
    )


def _chunk_w(w):
    dout, din = w.shape
    return w.T.reshape(din // F, F, dout)


def _src_offsets(src_r, chunk_ids):
    cs = jnp.asarray(chunk_ids, dtype=jnp.int32) * NP
    return src_r[None] + cs.reshape(-1, 1, 1, 1, 1)


def kernel(x, edge_index, Wl0, bl0, Wr0, Wl1, bl1, Wr1, Wl2, bl2, Wr2):
    src = edge_index[0].reshape(NC, NS, NB, EB)
    dst = edge_index[1].reshape(NC, NS, NB, EB)

    xp = jnp.pad(x, ((0, NP - N), (0, 0)))
    x_cm = jnp.stack([xp[:, :F], xp[:, F:]])
    zrows = jnp.zeros((RPT, F), jnp.float32)

    h_flat0 = jnp.concatenate(
        [x_cm.reshape(2 * NP, F), jnp.ones((NP, F), jnp.float32)])
    s_all = _make_agg(3)(h_flat0, _src_offsets(src, [0, 1, 2]), dst, zrows)
    wrp0 = _make_wr(2, 4)(x_cm, _chunk_w(Wr0), bl0.reshape(1, -1))

    h1, wrp1 = _make_combine(2, 4, 3, 2, 512, jax.nn.relu, True)(
        s_all, s_all, wrp0, _chunk_w(Wl0),
        Wr1.T, bl1.reshape(1, -1))

    s_1 = _make_agg(4)(h1.reshape(4 * NP, F),
                       _src_offsets(src, [0, 1, 2, 3]), dst, zrows)

    w2cat = jnp.concatenate([Wl2.T, Wr2.T], axis=1)
    b2cat = jnp.concatenate(
        [jnp.zeros((256,), jnp.float32), bl2]).reshape(1, -1)
    (sec,) = _make_combine(4, 4, 3, 2, 512, jax.nn.relu, False)(
        s_1, s_all, wrp1, _chunk_w(Wl1), w2cat, b2cat)

    s_p = _make_agg(2)(sec.reshape(4 * NP, F),
                       _src_offsets(src, [0, 1]), dst, zrows)

    h, sig = _make_final(3, 2)(s_p, s_all, sec)
    return (h[:N], sig[:N])


# speedup vs baseline: 6.2771x; 1.0052x over previous
import jax
import jax.numpy as jnp
from jax import lax
from jax.experimental import pallas as pl
from jax.experimental.pallas import tpu as pltpu
from jax.experimental.pallas import tpu_sc as plsc

N = 10000
NP = 10240
E = 160000
NC = 2
NS = 16
NB = 40
EB = 125
RPT = NP // NS
F = 128
BM = 1024
MB = NP // BM


def _sc_mesh():
    return plsc.VectorSubcoreMesh(core_axis_name="c", subcore_axis_name="s")


def _make_agg(C: int):
    out_type = jax.ShapeDtypeStruct((NC, C, NP, F), jnp.float32)

    scratch = [
        pltpu.VMEM_SHARED((NP, F), jnp.float32),
        pltpu.VMEM((NB, EB), jnp.int32),
        pltpu.VMEM((NB, EB), jnp.int32),
        pltpu.VMEM((EB, F), jnp.float32),
        pltpu.VMEM((EB, F), jnp.float32),
        pltpu.SemaphoreType.DMA,
        pltpu.SemaphoreType.DMA,
    ]

    def body(h_flat, src_off, dst_r, zrows, s_out, accum, srcv, dstv,
             rows0, rows1, sem0, sem1):
        ci = lax.axis_index("c")
        si = lax.axis_index("s")
        tbase = si * RPT

        pltpu.sync_copy(dst_r.at[ci, si], dstv)

        def gather(b, rbuf, sem):
            pltpu.async_copy(h_flat.at[srcv.at[b]], rbuf, sem)

        def gwait(b, rbuf, sem):
            pltpu.make_async_copy(h_flat.at[srcv.at[b]], rbuf, sem).wait()

        def chunk_pass(c, _):
            pltpu.sync_copy(src_off.at[c, ci, si], srcv)
            pltpu.sync_copy(zrows, accum.at[pl.ds(tbase, RPT)])
            plsc.subcore_barrier()

            gather(0, rows0, sem0)

            def pair(g, __):
                b0 = 2 * g
                b1 = 2 * g + 1
                gwait(b0, rows0, sem0)
                gather(b1, rows1, sem1)
                pltpu.sync_copy(rows0, accum.at[dstv.at[b0]], add=True)
                gwait(b1, rows1, sem1)

                @pl.when(g < NB // 2 - 1)
                def _():
                    gather(b0 + 2, rows0, sem0)

                pltpu.sync_copy(rows1, accum.at[dstv.at[b1]], add=True)
                return 0

            lax.fori_loop(0, NB // 2, pair, 0)
            plsc.subcore_barrier()
            pltpu.sync_copy(accum.at[pl.ds(tbase, RPT)],
                            s_out.at[ci, c, pl.ds(tbase, RPT)])
            plsc.subcore_barrier()
            return 0

        lax.fori_loop(0, C, chunk_pass, 0)

    return pl.kernel(
        body,
        out_type=out_type,
        mesh=_sc_mesh(),
        scratch_types=scratch,
    )


_TC_PARAMS = dict(
    compiler_params=pltpu.CompilerParams(
        dimension_semantics=("parallel", "arbitrary")),
)


def _make_wr(C_h: int, C_out: int):
    K = C_h
    DOUT = C_out * F

    def body(h_ref, wr_ref, b_ref, out_ref):
        k = pl.program_id(1)

        @pl.when(k == 0)
        def _():
            out_ref[...] = jnp.zeros(out_ref.shape, out_ref.dtype)

        t = jnp.dot(h_ref[0].astype(jnp.bfloat16), wr_ref[0],
                    preferred_element_type=jnp.float32)
        for c in range(C_out):
            out_ref[c] += t[:, c * F:(c + 1) * F]

        @pl.when(k == K - 1)
        def _():
            for c in range(C_out):
                out_ref[c] += b_ref[0, c * F:(c + 1) * F][None, :]

    return pl.pallas_call(
        body,
        grid=(MB, K),
        in_specs=[
            pl.BlockSpec((1, BM, F), lambda m, k: (k, m, 0)),
            pl.BlockSpec((1, F, DOUT), lambda m, k: (k, 0, 0)),
            pl.BlockSpec((1, DOUT), lambda m, k: (0, 0)),
        ],
        out_specs=pl.BlockSpec((C_out, BM, F), lambda m, k: (0, m, 0)),
        out_shape=jax.ShapeDtypeStruct((C_out, NP, F), jnp.float32),
        **_TC_PARAMS,
    )


def _make_combine(C_a: int, C_out: int, cnt_cs: int, cnt_idx: int,
                  sec_dout: int, act, emit_h: bool):
    K = C_a
    DOUT = C_out * F
    SC_CH = sec_dout // F if sec_dout else 0

    def body(*refs):
        it = iter(refs)
        s_ref = next(it)
        cnt_ref = next(it)
        wrp_ref = next(it)
        wl_ref = next(it)
        w2_ref = next(it) if sec_dout else None
        b2_ref = next(it) if sec_dout else None
        h_ref = next(it) if emit_h else None
        sec_ref = next(it) if sec_dout else None
        acc_ref = next(it)

        k = pl.program_id(1)

        @pl.when(k == 0)
        def _():
            acc_ref[...] = jnp.zeros(acc_ref.shape, acc_ref.dtype)

        cnt = cnt_ref[0, 0, :, 0:1] + cnt_ref[1, 0, :, 0:1]
        inv = 1.0 / jnp.maximum(cnt, 1.0)
        a = (s_ref[0, 0] + s_ref[1, 0]) * inv
        acc_ref[...] += jnp.dot(a.astype(jnp.bfloat16), wl_ref[0],
                                preferred_element_type=jnp.float32)

        @pl.when(k == K - 1)
        def _():
            parts = []
            for c in range(C_out):
                v = acc_ref[:, c * F:(c + 1) * F] + wrp_ref[c]
                if act is not None:
                    v = act(v)
                if emit_h:
                    h_ref[c] = v
                parts.append(v)
            if sec_dout:
                vfull = jnp.concatenate(parts, axis=1)
                t2 = jnp.dot(vfull.astype(jnp.bfloat16), w2_ref[...],
                             preferred_element_type=jnp.float32)
                for c2 in range(SC_CH):
                    sec_ref[c2] = (t2[:, c2 * F:(c2 + 1) * F]
                                   + b2_ref[0, c2 * F:(c2 + 1) * F][None, :])

    in_specs = [
        pl.BlockSpec((NC, 1, BM, F), lambda m, k: (0, k, m, 0)),
        pl.BlockSpec((NC, 1, BM, F), lambda m, k: (0, cnt_idx, m, 0)),
        pl.BlockSpec((C_out, BM, F), lambda m, k: (0, m, 0)),
        pl.BlockSpec((1, F, DOUT), lambda m, k: (k, 0, 0)),
    ]
    if sec_dout:
        in_specs.append(pl.BlockSpec((DOUT, sec_dout), lambda m, k: (0, 0)))
        in_specs.append(pl.BlockSpec((1, sec_dout), lambda m, k: (0, 0)))

    out_specs = []
    out_shape = []
    if emit_h:
        out_specs.append(pl.BlockSpec((C_out, BM, F), lambda m, k: (0, m, 0)))
        out_shape.append(jax.ShapeDtypeStruct((C_out, NP, F), jnp.float32))
    if sec_dout:
        out_specs.append(pl.BlockSpec((SC_CH, BM, F), lambda m, k: (0, m, 0)))
        out_shape.append(jax.ShapeDtypeStruct((SC_CH, NP, F), jnp.float32))

    return pl.pallas_call(
        body,
        grid=(MB, K),
        in_specs=in_specs,
        out_specs=out_specs,
        out_shape=out_shape,
        scratch_shapes=[pltpu.VMEM((BM, DOUT), jnp.float32)],
        **_TC_PARAMS,
    )


def _make_final(cnt_cs: int, cnt_idx: int):
    DOUT = 2 * F

    def body(sp_ref, cnt_ref, w2_ref, o1_ref, o2_ref):
        cnt = cnt_ref[0, 0, :, 0:1] + cnt_ref[1, 0, :, 0:1]
        inv = 1.0 / jnp.maximum(cnt, 1.0)
        mean = jnp.concatenate(
            [(sp_ref[0, c] + sp_ref[1, c]) * inv for c in range(2)], axis=1)
        wrp2 = jnp.concatenate([w2_ref[c] for c in range(2)], axis=1)
        v = mean + wrp2
        o1_ref[...] = v
        o2_ref[...] = jax.nn.sigmoid(v)

    return pl.pallas_call(
        body,
        grid=(MB,),
        in_specs=[
            pl.BlockSpec((NC, 2, BM, F), lambda m: (0, 0, m, 0)),
            pl.BlockSpec((NC, 1, BM, F), lambda m: (0, cnt_idx, m, 0)),
            pl.BlockSpec((2, BM, F), lambda m: (1, m, 0)),
        ],
        out_specs=[
            pl.BlockSpec((BM, DOUT), lambda m: (m, 0)),
            pl.BlockSpec((BM, DOUT), lambda m: (m, 0)),
        ],
        out_shape=[
            jax.ShapeDtypeStruct((NP, DOUT), jnp.float32),
            jax.ShapeDtypeStruct((NP, DOUT), jnp.float32),
        ],
        compiler_params=pltpu.CompilerParams(
            dimension_semantics=("parallel",)),
    )


def _chunk_w(w):
    dout, din = w.shape
    return w.T.reshape(din // F, F, dout).astype(jnp.bfloat16)


def _src_offsets(src_r, chunk_ids):
    cs = jnp.asarray(chunk_ids, dtype=jnp.int32) * NP
    return src_r[None] + cs.reshape(-1, 1, 1, 1, 1)


def kernel(x, edge_index, Wl0, bl0, Wr0, Wl1, bl1, Wr1, Wl2, bl2, Wr2):
    src = edge_index[0].reshape(NC, NS, NB, EB)
    dst = edge_index[1].reshape(NC, NS, NB, EB)

    xp = jnp.pad(x, ((0, NP - N), (0, 0)))
    x_cm = jnp.stack([xp[:, :F], xp[:, F:]])
    zrows = jnp.zeros((RPT, F), jnp.float32)

    h_flat0 = jnp.concatenate(
        [x_cm.reshape(2 * NP, F), jnp.ones((NP, F), jnp.float32)])
    s_all = _make_agg(3)(h_flat0, _src_offsets(src, [0, 1, 2]), dst, zrows)
    wrp0 = _make_wr(2, 4)(x_cm, _chunk_w(Wr0), bl0.reshape(1, -1))

    h1, wrp1 = _make_combine(2, 4, 3, 2, 512, jax.nn.relu, True)(
        s_all, s_all, wrp0, _chunk_w(Wl0),
        Wr1.T.astype(jnp.bfloat16), bl1.reshape(1, -1))

    s_1 = _make_agg(4)(h1.reshape(4 * NP, F),
                       _src_offsets(src, [0, 1, 2, 3]), dst, zrows)

    w2cat = jnp.concatenate(
        [Wl2.T, Wr2.T], axis=1).astype(jnp.bfloat16)
    b2cat = jnp.concatenate(
        [jnp.zeros((256,), jnp.float32), bl2]).reshape(1, -1)
    (sec,) = _make_combine(4, 4, 3, 2, 512, jax.nn.relu, False)(
        s_1, s_all, wrp1, _chunk_w(Wl1), w2cat, b2cat)

    s_p = _make_agg(2)(sec.reshape(4 * NP, F),
                       _src_offsets(src, [0, 1]), dst, zrows)

    h, sig = _make_final(3, 2)(s_p, s_all, sec)
    return (h[:N], sig[:N])
